# baseline (speedup 1.0000x reference)
import functools

import jax
import jax.numpy as jnp
from jax import lax
from jax.experimental import pallas as pl
from jax.experimental.pallas import tpu as pltpu
from jax.experimental.pallas import tpu_sc as plsc

N = 10000
E = 320000
H = 128
U = 16
B = 128

_info = plsc.get_sparse_core_info()
NC = _info.num_cores
NS = _info.num_subcores
L = _info.num_lanes
NW = NC * NS

TCBLK = 1024
TC_STEPS = 120
E_TC = TC_STEPS * TCBLK
E_SC = E - E_TC
NPAD = 80 * 128

E_PER_W = E_SC // NW
ECHUNK = 256
E_FULL = E_PER_W // ECHUNK
E_TAIL = E_PER_W - E_FULL * ECHUNK
NBUF = 3

NCHUNK = 128
N_FULL = N // NCHUNK
N_TAIL = N - N_FULL * NCHUNK
N_ITERS = (N_FULL + NW - 1) // NW

STRIPE = B // NS

_mesh = plsc.VectorSubcoreMesh(core_axis_name="c", subcore_axis_name="s")


@functools.partial(
    pl.kernel,
    mesh=_mesh,
    compiler_params=pltpu.CompilerParams(needs_layout_passes=False),
    out_type=[
        jax.ShapeDtypeStruct((NC, B, H), jnp.float32),
        jax.ShapeDtypeStruct((NC, B, H), jnp.float32),
        jax.ShapeDtypeStruct((NC * 8, B), jnp.float32),
        jax.ShapeDtypeStruct((NC * 8, B), jnp.float32),
    ],
    scratch_types=[
        pltpu.VMEM((N,), jnp.int32),
        pltpu.VMEM((NBUF, ECHUNK, H), jnp.float32),
        pltpu.VMEM((E_PER_W,), jnp.int32),
        pltpu.VMEM((2 * NBUF, 128), jnp.int32),
        pltpu.VMEM((1, L), jnp.int32),
        pltpu.VMEM((1, B), jnp.int32),
        pltpu.VMEM((B,), jnp.float32),
        pltpu.VMEM((B,), jnp.float32),
        pltpu.VMEM((B,), jnp.float32),
        pltpu.VMEM((STRIPE, H), jnp.float32),
        pltpu.VMEM_SHARED((B, H), jnp.float32),
        pltpu.VMEM_SHARED((B, H), jnp.float32),
        pltpu.VMEM_SHARED((B,), jnp.float32),
        pltpu.VMEM_SHARED((B,), jnp.float32),
        pltpu.SemaphoreType.DMA,
        pltpu.SemaphoreType.DMA,
        pltpu.SemaphoreType.DMA,
        pltpu.SemaphoreType.DMA,
        pltpu.SemaphoreType.DMA,
        pltpu.SemaphoreType.DMA,
        pltpu.SemaphoreType.DMA,
    ],
)
def _sc_aggregate(x_hbm, ei_hbm, ea_hbm, batch_hbm,
                  nsum_hbm, esum_hbm, ncnt_hbm, ecnt_hbm,
                  batch_v, rows_v, dest_v, seg_v, segt_v, ident_v,
                  ecnt_v, ncnt_v, zeros1_v, zeros_v,
                  nsum_sh, esum_sh, ncnt_sh, ecnt_sh,
                  sem_in0, sem_in1, sem_in2, sem_id,
                  sem_sc0, sem_sc1, sem_sc2):
    c = lax.axis_index("c")
    s = lax.axis_index("s")
    wid = s * NC + c
    sem_in = (sem_in0, sem_in1, sem_in2)
    sem_sc = (sem_sc0, sem_sc1, sem_sc2)
    ebase = E_TC + wid * E_PER_W

    def start_in(j, b):
        off = ebase + j * ECHUNK
        pltpu.async_copy(ea_hbm.at[pl.ds(off, ECHUNK), :], rows_v.at[b],
                         sem_in[b])

    pltpu.async_copy(ei_hbm.at[pl.ds(E + ebase, E_PER_W)], dest_v, sem_id)
    start_in(0, 0)
    start_in(1, 1)
    start_in(2, 2)

    zero16 = jnp.zeros((L,), jnp.float32)
    i16 = lax.broadcasted_iota(jnp.int32, (L,), 0)
    for r in range(STRIPE):
        for cc in range(H // L):
            zeros_v[r, pl.ds(cc * L, L)] = zero16
    for cc in range(B // L):
        zeros1_v[pl.ds(cc * L, L)] = zero16
        ecnt_v[pl.ds(cc * L, L)] = zero16
        ncnt_v[pl.ds(cc * L, L)] = zero16
        ident_v[0, pl.ds(cc * L, L)] = i16 + cc * L

    st = pl.multiple_of(s * STRIPE, STRIPE)
    pltpu.sync_copy(zeros_v, nsum_sh.at[pl.ds(st, STRIPE)])
    pltpu.sync_copy(zeros_v, esum_sh.at[pl.ds(st, STRIPE)])

    @pl.when(s == 0)
    def _():
        pltpu.sync_copy(zeros1_v, ncnt_sh)
        pltpu.sync_copy(zeros1_v, ecnt_sh)

    pltpu.sync_copy(batch_hbm, batch_v)
    plsc.subcore_barrier()
    pltpu.make_async_copy(ei_hbm.at[pl.ds(E, E_PER_W)], dest_v, sem_id).wait()

    def wait_in(b):
        pltpu.make_async_copy(ea_hbm.at[pl.ds(0, ECHUNK), :], rows_v.at[b],
                              sem_in[b]).wait()

    def gather_ids(m, b):
        one16 = jnp.ones((L,), jnp.float32)
        for r in range(ECHUNK // 128):
            row = 2 * b + r
            for i in range(128 // L):
                d16 = dest_v[pl.ds(m * ECHUNK + r * 128 + i * L, L)]
                s16 = plsc.load_gather(batch_v, [d16])
                seg_v[row, pl.ds(i * L, L)] = s16
                plsc.addupdate_scatter(ecnt_v, [s16], one16)

    def start_scat(b):
        for r in range(ECHUNK // 128):
            row = 2 * b + r
            pltpu.async_copy(rows_v.at[b, pl.ds(r * 128, 128)],
                             esum_sh.at[seg_v.at[row]], sem_sc[b], add=True)

    def wait_scat(b):
        for r in range(ECHUNK // 128):
            row = 2 * b + r
            pltpu.make_async_copy(rows_v.at[b, pl.ds(r * 128, 128)],
                                  esum_sh.at[seg_v.at[row]], sem_sc[b]).wait()

    def step(m, b):
        wait_in(b)
        gather_ids(m, b)
        start_scat(b)

    for m in range(E_FULL):
        step(m, m % NBUF)
        if m >= 1 and m + 2 < E_FULL:
            pb = (m + 2) % NBUF
            wait_scat(pb)
            start_in(m + 2, pb)

    wait_scat(0)
    offt = ebase + E_FULL * ECHUNK
    pltpu.sync_copy(ea_hbm.at[pl.ds(offt, E_TAIL), :],
                    rows_v.at[0, pl.ds(0, E_TAIL)])
    d16 = dest_v[pl.ds(E_FULL * ECHUNK, L)]
    s16 = plsc.load_gather(batch_v, [d16])
    segt_v[0, pl.ds(0, L)] = s16
    plsc.addupdate_scatter(ecnt_v, [s16], jnp.ones((L,), jnp.float32))
    pltpu.sync_copy(rows_v.at[0, pl.ds(0, E_TAIL)],
                    esum_sh.at[segt_v.at[0]], add=True)
    wait_scat(1)
    wait_scat(2)

    def n_start_in(i, b):
        k = i * NW + wid
        off = k * NCHUNK
        pltpu.async_copy(batch_hbm.at[pl.ds(off, NCHUNK)], seg_v.at[2 * b],
                         sem_in[b])
        pltpu.async_copy(x_hbm.at[pl.ds(off, NCHUNK), :],
                         rows_v.at[b, pl.ds(0, NCHUNK)], sem_in[b])

    def n_wait_in(b):
        pltpu.make_async_copy(batch_hbm.at[pl.ds(0, NCHUNK)], seg_v.at[2 * b],
                              sem_in[b]).wait()
        pltpu.make_async_copy(x_hbm.at[pl.ds(0, NCHUNK), :],
                              rows_v.at[b, pl.ds(0, NCHUNK)], sem_in[b]).wait()

    def n_scat(b):
        one16 = jnp.ones((L,), jnp.float32)
        pltpu.async_copy(rows_v.at[b, pl.ds(0, NCHUNK)],
                         nsum_sh.at[seg_v.at[2 * b]], sem_sc[b], add=True)
        for i in range(NCHUNK // L):
            s16 = seg_v[2 * b, pl.ds(i * L, L)]
            plsc.addupdate_scatter(ncnt_v, [s16], one16)

    def n_wait_scat(b):
        pltpu.make_async_copy(rows_v.at[b, pl.ds(0, NCHUNK)],
                              nsum_sh.at[seg_v.at[2 * b]], sem_sc[b]).wait()

    for i in range(N_ITERS):
        @pl.when(i * NW + wid < N_FULL)
        def _(i=i):
            n_start_in(i, i)

    for i in range(N_ITERS):
        @pl.when(i * NW + wid < N_FULL)
        def _(i=i):
            n_wait_in(i)
            n_scat(i)

    @pl.when(wid == NW - 1)
    def _():
        offn = N_FULL * NCHUNK
        pltpu.sync_copy(batch_hbm.at[pl.ds(offn, N_TAIL)], segt_v.at[0])
        pltpu.sync_copy(x_hbm.at[pl.ds(offn, N_TAIL), :],
                        rows_v.at[1, pl.ds(128, N_TAIL)])
        pltpu.sync_copy(rows_v.at[1, pl.ds(128, N_TAIL)],
                        nsum_sh.at[segt_v.at[0]], add=True)
        t16 = segt_v[0, pl.ds(0, L)]
        plsc.addupdate_scatter(ncnt_v, [t16], jnp.ones((L,), jnp.float32))

    for i in range(N_ITERS):
        @pl.when(i * NW + wid < N_FULL)
        def _(i=i):
            n_wait_scat(i)

    pltpu.sync_copy(ecnt_v, ecnt_sh.at[ident_v.at[0]], add=True)
    pltpu.sync_copy(ncnt_v, ncnt_sh.at[ident_v.at[0]], add=True)

    plsc.subcore_barrier()

    pltpu.sync_copy(nsum_sh.at[pl.ds(st, STRIPE)], nsum_hbm.at[c, pl.ds(st, STRIPE)])
    pltpu.sync_copy(esum_sh.at[pl.ds(st, STRIPE)], esum_hbm.at[c, pl.ds(st, STRIPE)])

    @pl.when(s == 0)
    def _():
        crow = pl.multiple_of(c * 8, 8)
        pltpu.sync_copy(ncnt_sh, ncnt_hbm.at[crow])
        pltpu.sync_copy(ecnt_sh, ecnt_hbm.at[crow])


def _tc_edge_body(batch_ref, dest_ref, attr_ref, psum_ref, pcnt_ref,
                  bnd_ref, bnd2_ref):
    j = pl.program_id(0)

    @pl.when(j == 0)
    def _():
        bt = batch_ref[...][:, :, None]
        lanes = lax.broadcasted_iota(jnp.int32, (1, 1, B), 2)
        bnd_ref[...] = jnp.sum((bt < lanes).astype(jnp.int32),
                               axis=(0, 1))[None, :]
        bnd2_ref[...] = jnp.sum((bt <= lanes).astype(jnp.int32),
                                axis=(0, 1))[None, :]
        psum_ref[...] = jnp.zeros_like(psum_ref)
        pcnt_ref[...] = jnp.zeros_like(pcnt_ref)

    d3 = dest_ref[...][:, :, None]
    b1 = bnd_ref[...][None, :, :]
    b2 = bnd2_ref[...][None, :, :]
    oh = ((d3 >= b1).astype(jnp.float32)
          - (d3 >= b2).astype(jnp.float32)).reshape(TCBLK, B)
    psum_ref[...] += lax.dot_general(
        oh, attr_ref[...], (((0,), (0,)), ((), ())),
        preferred_element_type=jnp.float32)
    pcnt_ref[...] += jnp.sum(oh, axis=0, keepdims=True)


def _tc_edge_aggregate(batch_pad, dest_tc, edge_attr):
    return pl.pallas_call(
        _tc_edge_body,
        grid=(TC_STEPS,),
        in_specs=[
            pl.BlockSpec((NPAD // 128, 128), lambda j: (0, 0)),
            pl.BlockSpec((TCBLK // 128, 128), lambda j: (j, 0)),
            pl.BlockSpec((TCBLK, H), lambda j: (j, 0)),
        ],
        out_specs=[
            pl.BlockSpec((B, H), lambda j: (0, 0)),
            pl.BlockSpec((1, B), lambda j: (0, 0)),
        ],
        out_shape=[
            jax.ShapeDtypeStruct((B, H), jnp.float32),
            jax.ShapeDtypeStruct((1, B), jnp.float32),
        ],
        scratch_shapes=[
            pltpu.VMEM((1, B), jnp.int32),
            pltpu.VMEM((1, B), jnp.int32),
        ],
    )(batch_pad, dest_tc, edge_attr)


def _mlp_body(nsum_ref, esum_ref, ncnt_ref, ecnt_ref, tces_ref, tcec_ref,
              u_ref, w1_ref, b1_ref, w2_ref, b2_ref, o_ref):
    dn = (((1,), (1,)), ((), ()))
    ns = nsum_ref[0] + nsum_ref[1]
    es = esum_ref[0] + esum_ref[1] + tces_ref[...]
    nc = ncnt_ref[0:1] + ncnt_ref[8:9]
    ec = ecnt_ref[0:1] + ecnt_ref[8:9] + tcec_ref[...]
    ir = lax.broadcasted_iota(jnp.int32, (B, B), 0)
    ic = lax.broadcasted_iota(jnp.int32, (B, B), 1)
    eye = jnp.where(ir == ic, 1.0, 0.0).astype(jnp.float32)
    ninv_d = eye * (1.0 / jnp.maximum(nc, 1.0))
    einv_d = eye * (1.0 / jnp.maximum(ec, 1.0))
    node_mean = lax.dot_general(ninv_d, ns, (((1,), (0,)), ((), ())),
                                preferred_element_type=jnp.float32)
    edge_mean = lax.dot_general(einv_d, es, (((1,), (0,)), ((), ())),
                                preferred_element_type=jnp.float32)
    h = (lax.dot_general(u_ref[...], w1_ref[:, 0:U], dn,
                         preferred_element_type=jnp.float32)
         + lax.dot_general(node_mean, w1_ref[:, U:U + H], dn,
                           preferred_element_type=jnp.float32)
         + lax.dot_general(edge_mean, w1_ref[:, U + H:U + 2 * H], dn,
                           preferred_element_type=jnp.float32)
         + b1_ref[...])
    h = jnp.maximum(h, 0.0)
    o_ref[...] = (lax.dot_general(h, w2_ref[...], dn,
                                  preferred_element_type=jnp.float32)
                  + b2_ref[...])


def kernel(x, edge_index, edge_attr, u, batch, W1, b1, W2, b2):
    ei_flat = edge_index.reshape(-1)
    nsum, esum, ncnt, ecnt = _sc_aggregate(x, ei_flat, edge_attr, batch)
    batch_pad = jnp.concatenate(
        [batch, jnp.full((NPAD - N,), B, jnp.int32)]).reshape(NPAD // 128, 128)
    dest_tc = ei_flat[E:E + E_TC].reshape(E_TC // 128, 128)
    tces, tcec = _tc_edge_aggregate(batch_pad, dest_tc, edge_attr)
    return pl.pallas_call(
        _mlp_body,
        out_shape=jax.ShapeDtypeStruct((B, H), jnp.float32),
    )(nsum, esum, ncnt, ecnt, tces, tcec, u,
      W1, b1.reshape(1, H), W2, b2.reshape(1, H))


# speedup vs baseline: 1.1868x; 1.1868x over previous
import functools

import jax
import jax.numpy as jnp
from jax import lax
from jax.experimental import pallas as pl
from jax.experimental.pallas import tpu as pltpu
from jax.experimental.pallas import tpu_sc as plsc

N = 10000
E = 320000
H = 128
U = 16
B = 128

_info = plsc.get_sparse_core_info()
NC = _info.num_cores
NS = _info.num_subcores
L = _info.num_lanes
NW = NC * NS

TCBLK = 1024
TC_STEPS = 64
E_TC = TC_STEPS * TCBLK
E_SC = E - E_TC
NPAD = 80 * 128

E_PER_W = E_SC // NW
ECHUNK = 256
E_FULL = E_PER_W // ECHUNK
E_TAIL = E_PER_W - E_FULL * ECHUNK
NBUF = 3

NCHUNK = 128
N_FULL = N // NCHUNK
N_TAIL = N - N_FULL * NCHUNK
N_ITERS = (N_FULL + NW - 1) // NW

STRIPE = B // NS

_mesh = plsc.VectorSubcoreMesh(core_axis_name="c", subcore_axis_name="s")


@functools.partial(
    pl.kernel,
    mesh=_mesh,
    compiler_params=pltpu.CompilerParams(needs_layout_passes=False),
    out_type=[
        jax.ShapeDtypeStruct((NC, B, H), jnp.float32),
        jax.ShapeDtypeStruct((NC, B, H), jnp.float32),
        jax.ShapeDtypeStruct((NC * 8, B), jnp.float32),
        jax.ShapeDtypeStruct((NC * 8, B), jnp.float32),
    ],
    scratch_types=[
        pltpu.VMEM((N,), jnp.int32),
        pltpu.VMEM((NBUF, ECHUNK, H), jnp.float32),
        pltpu.VMEM((E_PER_W,), jnp.int32),
        pltpu.VMEM((2 * NBUF, 128), jnp.int32),
        pltpu.VMEM((1, L), jnp.int32),
        pltpu.VMEM((1, B), jnp.int32),
        pltpu.VMEM((B,), jnp.float32),
        pltpu.VMEM((B,), jnp.float32),
        pltpu.VMEM((B,), jnp.float32),
        pltpu.VMEM((STRIPE, H), jnp.float32),
        pltpu.VMEM_SHARED((B, H), jnp.float32),
        pltpu.VMEM_SHARED((B, H), jnp.float32),
        pltpu.VMEM_SHARED((B,), jnp.float32),
        pltpu.VMEM_SHARED((B,), jnp.float32),
        pltpu.SemaphoreType.DMA,
        pltpu.SemaphoreType.DMA,
        pltpu.SemaphoreType.DMA,
        pltpu.SemaphoreType.DMA,
        pltpu.SemaphoreType.DMA,
        pltpu.SemaphoreType.DMA,
        pltpu.SemaphoreType.DMA,
    ],
)
def _sc_aggregate(x_hbm, ei_hbm, ea_hbm, batch_hbm,
                  nsum_hbm, esum_hbm, ncnt_hbm, ecnt_hbm,
                  batch_v, rows_v, dest_v, seg_v, segt_v, ident_v,
                  ecnt_v, ncnt_v, zeros1_v, zeros_v,
                  nsum_sh, esum_sh, ncnt_sh, ecnt_sh,
                  sem_in0, sem_in1, sem_in2, sem_id,
                  sem_sc0, sem_sc1, sem_sc2):
    c = lax.axis_index("c")
    s = lax.axis_index("s")
    wid = s * NC + c
    sem_in = (sem_in0, sem_in1, sem_in2)
    sem_sc = (sem_sc0, sem_sc1, sem_sc2)
    ebase = E_TC + wid * E_PER_W

    def start_in(j, b):
        off = ebase + j * ECHUNK
        pltpu.async_copy(ea_hbm.at[pl.ds(off, ECHUNK), :], rows_v.at[b],
                         sem_in[b])

    pltpu.async_copy(ei_hbm.at[pl.ds(E + ebase, E_PER_W)], dest_v, sem_id)
    start_in(0, 0)
    start_in(1, 1)
    start_in(2, 2)

    zero16 = jnp.zeros((L,), jnp.float32)
    i16 = lax.broadcasted_iota(jnp.int32, (L,), 0)
    for r in range(STRIPE):
        for cc in range(H // L):
            zeros_v[r, pl.ds(cc * L, L)] = zero16
    for cc in range(B // L):
        zeros1_v[pl.ds(cc * L, L)] = zero16
        ecnt_v[pl.ds(cc * L, L)] = zero16
        ncnt_v[pl.ds(cc * L, L)] = zero16
        ident_v[0, pl.ds(cc * L, L)] = i16 + cc * L

    st = pl.multiple_of(s * STRIPE, STRIPE)
    pltpu.sync_copy(zeros_v, nsum_sh.at[pl.ds(st, STRIPE)])
    pltpu.sync_copy(zeros_v, esum_sh.at[pl.ds(st, STRIPE)])

    @pl.when(s == 0)
    def _():
        pltpu.sync_copy(zeros1_v, ncnt_sh)
        pltpu.sync_copy(zeros1_v, ecnt_sh)

    pltpu.sync_copy(batch_hbm, batch_v)
    plsc.subcore_barrier()
    pltpu.make_async_copy(ei_hbm.at[pl.ds(E, E_PER_W)], dest_v, sem_id).wait()

    def wait_in(b):
        pltpu.make_async_copy(ea_hbm.at[pl.ds(0, ECHUNK), :], rows_v.at[b],
                              sem_in[b]).wait()

    def gather_ids(m, b):
        one16 = jnp.ones((L,), jnp.float32)
        for r in range(ECHUNK // 128):
            row = 2 * b + r
            for i in range(128 // L):
                d16 = dest_v[pl.ds(m * ECHUNK + r * 128 + i * L, L)]
                s16 = plsc.load_gather(batch_v, [d16])
                seg_v[row, pl.ds(i * L, L)] = s16
                plsc.addupdate_scatter(ecnt_v, [s16], one16)

    def start_scat(b):
        for r in range(ECHUNK // 128):
            row = 2 * b + r
            pltpu.async_copy(rows_v.at[b, pl.ds(r * 128, 128)],
                             esum_sh.at[seg_v.at[row]], sem_sc[b], add=True)

    def wait_scat(b):
        for r in range(ECHUNK // 128):
            row = 2 * b + r
            pltpu.make_async_copy(rows_v.at[b, pl.ds(r * 128, 128)],
                                  esum_sh.at[seg_v.at[row]], sem_sc[b]).wait()

    def step(m, b):
        wait_in(b)
        gather_ids(m, b)
        start_scat(b)

    for m in range(E_FULL):
        step(m, m % NBUF)
        if m >= 1 and m + 2 < E_FULL:
            pb = (m + 2) % NBUF
            wait_scat(pb)
            start_in(m + 2, pb)

    wait_scat(0)
    offt = ebase + E_FULL * ECHUNK
    pltpu.sync_copy(ea_hbm.at[pl.ds(offt, E_TAIL), :],
                    rows_v.at[0, pl.ds(0, E_TAIL)])
    d16 = dest_v[pl.ds(E_FULL * ECHUNK, L)]
    s16 = plsc.load_gather(batch_v, [d16])
    segt_v[0, pl.ds(0, L)] = s16
    plsc.addupdate_scatter(ecnt_v, [s16], jnp.ones((L,), jnp.float32))
    pltpu.sync_copy(rows_v.at[0, pl.ds(0, E_TAIL)],
                    esum_sh.at[segt_v.at[0]], add=True)
    wait_scat(1)
    wait_scat(2)

    def n_start_in(i, b):
        k = i * NW + wid
        off = k * NCHUNK
        pltpu.async_copy(batch_hbm.at[pl.ds(off, NCHUNK)], seg_v.at[2 * b],
                         sem_in[b])
        pltpu.async_copy(x_hbm.at[pl.ds(off, NCHUNK), :],
                         rows_v.at[b, pl.ds(0, NCHUNK)], sem_in[b])

    def n_wait_in(b):
        pltpu.make_async_copy(batch_hbm.at[pl.ds(0, NCHUNK)], seg_v.at[2 * b],
                              sem_in[b]).wait()
        pltpu.make_async_copy(x_hbm.at[pl.ds(0, NCHUNK), :],
                              rows_v.at[b, pl.ds(0, NCHUNK)], sem_in[b]).wait()

    def n_scat(b):
        one16 = jnp.ones((L,), jnp.float32)
        pltpu.async_copy(rows_v.at[b, pl.ds(0, NCHUNK)],
                         nsum_sh.at[seg_v.at[2 * b]], sem_sc[b], add=True)
        for i in range(NCHUNK // L):
            s16 = seg_v[2 * b, pl.ds(i * L, L)]
            plsc.addupdate_scatter(ncnt_v, [s16], one16)

    def n_wait_scat(b):
        pltpu.make_async_copy(rows_v.at[b, pl.ds(0, NCHUNK)],
                              nsum_sh.at[seg_v.at[2 * b]], sem_sc[b]).wait()

    for i in range(N_ITERS):
        @pl.when(i * NW + wid < N_FULL)
        def _(i=i):
            n_start_in(i, i)

    for i in range(N_ITERS):
        @pl.when(i * NW + wid < N_FULL)
        def _(i=i):
            n_wait_in(i)
            n_scat(i)

    @pl.when(wid == NW - 1)
    def _():
        offn = N_FULL * NCHUNK
        pltpu.sync_copy(batch_hbm.at[pl.ds(offn, N_TAIL)], segt_v.at[0])
        pltpu.sync_copy(x_hbm.at[pl.ds(offn, N_TAIL), :],
                        rows_v.at[1, pl.ds(128, N_TAIL)])
        pltpu.sync_copy(rows_v.at[1, pl.ds(128, N_TAIL)],
                        nsum_sh.at[segt_v.at[0]], add=True)
        t16 = segt_v[0, pl.ds(0, L)]
        plsc.addupdate_scatter(ncnt_v, [t16], jnp.ones((L,), jnp.float32))

    for i in range(N_ITERS):
        @pl.when(i * NW + wid < N_FULL)
        def _(i=i):
            n_wait_scat(i)

    pltpu.sync_copy(ecnt_v, ecnt_sh.at[ident_v.at[0]], add=True)
    pltpu.sync_copy(ncnt_v, ncnt_sh.at[ident_v.at[0]], add=True)

    plsc.subcore_barrier()

    pltpu.sync_copy(nsum_sh.at[pl.ds(st, STRIPE)], nsum_hbm.at[c, pl.ds(st, STRIPE)])
    pltpu.sync_copy(esum_sh.at[pl.ds(st, STRIPE)], esum_hbm.at[c, pl.ds(st, STRIPE)])

    @pl.when(s == 0)
    def _():
        crow = pl.multiple_of(c * 8, 8)
        pltpu.sync_copy(ncnt_sh, ncnt_hbm.at[crow])
        pltpu.sync_copy(ecnt_sh, ecnt_hbm.at[crow])


def _tc_edge_body(batch_ref, dest_ref, attr_ref, psum_ref, pcnt_ref,
                  bnd_ref, bnd2_ref):
    j = pl.program_id(0)

    @pl.when(j == 0)
    def _():
        bt = batch_ref[...][:, :, None]
        lanes = lax.broadcasted_iota(jnp.int32, (1, 1, B), 2)
        bnd_ref[...] = jnp.sum((bt < lanes).astype(jnp.int32),
                               axis=(0, 1))[None, :]
        bnd2_ref[...] = jnp.sum((bt <= lanes).astype(jnp.int32),
                                axis=(0, 1))[None, :]
        psum_ref[...] = jnp.zeros_like(psum_ref)
        pcnt_ref[...] = jnp.zeros_like(pcnt_ref)

    d3 = dest_ref[...][:, :, None]
    b1 = bnd_ref[...][None, :, :]
    b2 = bnd2_ref[...][None, :, :]
    oh = ((d3 >= b1).astype(jnp.float32)
          - (d3 >= b2).astype(jnp.float32)).reshape(TCBLK, B)
    psum_ref[...] += lax.dot_general(
        oh, attr_ref[...], (((0,), (0,)), ((), ())),
        preferred_element_type=jnp.float32)
    pcnt_ref[...] += jnp.sum(oh, axis=0, keepdims=True)


def _tc_edge_aggregate(batch_pad, dest_tc, edge_attr):
    return pl.pallas_call(
        _tc_edge_body,
        grid=(TC_STEPS,),
        in_specs=[
            pl.BlockSpec((NPAD // 128, 128), lambda j: (0, 0)),
            pl.BlockSpec((TCBLK // 128, 128), lambda j: (j, 0)),
            pl.BlockSpec((TCBLK, H), lambda j: (j, 0)),
        ],
        out_specs=[
            pl.BlockSpec((B, H), lambda j: (0, 0)),
            pl.BlockSpec((1, B), lambda j: (0, 0)),
        ],
        out_shape=[
            jax.ShapeDtypeStruct((B, H), jnp.float32),
            jax.ShapeDtypeStruct((1, B), jnp.float32),
        ],
        scratch_shapes=[
            pltpu.VMEM((1, B), jnp.int32),
            pltpu.VMEM((1, B), jnp.int32),
        ],
    )(batch_pad, dest_tc, edge_attr)


def _mlp_body(nsum_ref, esum_ref, ncnt_ref, ecnt_ref, tces_ref, tcec_ref,
              u_ref, w1_ref, b1_ref, w2_ref, b2_ref, o_ref):
    dn = (((1,), (1,)), ((), ()))
    ns = nsum_ref[0] + nsum_ref[1]
    es = esum_ref[0] + esum_ref[1] + tces_ref[...]
    nc = ncnt_ref[0:1] + ncnt_ref[8:9]
    ec = ecnt_ref[0:1] + ecnt_ref[8:9] + tcec_ref[...]
    ir = lax.broadcasted_iota(jnp.int32, (B, B), 0)
    ic = lax.broadcasted_iota(jnp.int32, (B, B), 1)
    eye = jnp.where(ir == ic, 1.0, 0.0).astype(jnp.float32)
    ninv_d = eye * (1.0 / jnp.maximum(nc, 1.0))
    einv_d = eye * (1.0 / jnp.maximum(ec, 1.0))
    node_mean = lax.dot_general(ninv_d, ns, (((1,), (0,)), ((), ())),
                                preferred_element_type=jnp.float32)
    edge_mean = lax.dot_general(einv_d, es, (((1,), (0,)), ((), ())),
                                preferred_element_type=jnp.float32)
    h = (lax.dot_general(u_ref[...], w1_ref[:, 0:U], dn,
                         preferred_element_type=jnp.float32)
         + lax.dot_general(node_mean, w1_ref[:, U:U + H], dn,
                           preferred_element_type=jnp.float32)
         + lax.dot_general(edge_mean, w1_ref[:, U + H:U + 2 * H], dn,
                           preferred_element_type=jnp.float32)
         + b1_ref[...])
    h = jnp.maximum(h, 0.0)
    o_ref[...] = (lax.dot_general(h, w2_ref[...], dn,
                                  preferred_element_type=jnp.float32)
                  + b2_ref[...])


def kernel(x, edge_index, edge_attr, u, batch, W1, b1, W2, b2):
    ei_flat = edge_index.reshape(-1)
    nsum, esum, ncnt, ecnt = _sc_aggregate(x, ei_flat, edge_attr, batch)
    batch_pad = jnp.concatenate(
        [batch, jnp.full((NPAD - N,), B, jnp.int32)]).reshape(NPAD // 128, 128)
    dest_tc = ei_flat[E:E + E_TC].reshape(E_TC // 128, 128)
    tces, tcec = _tc_edge_aggregate(batch_pad, dest_tc, edge_attr)
    return pl.pallas_call(
        _mlp_body,
        out_shape=jax.ShapeDtypeStruct((B, H), jnp.float32),
    )(nsum, esum, ncnt, ecnt, tces, tcec, u,
      W1, b1.reshape(1, H), W2, b2.reshape(1, H))
